# baseline (speedup 1.0000x reference)
import math

import jax
import jax.numpy as jnp
from jax.experimental import pallas as pl
from jax.experimental.pallas import tpu as pltpu

_GROUP = 4
_BUDGET = 0.5


def _body(sal_ref, msk_ref, wt_ref, b_ref, out_ref, lasso_ref,
          acc_ref, mask_acc_ref, *, n_h, n_b, n_ch, n_px, k_drop):
    bi = pl.program_id(0)
    hi = pl.program_id(1)

    sal = sal_ref[0]
    m = msk_ref[0]
    part = jnp.sum(sal * m, axis=1, keepdims=True)
    mpart = jnp.sum(m, axis=1)

    @pl.when(hi == 0)
    def _init():
        acc_ref[bi] = part
        mask_acc_ref[pl.ds(bi, 1), :] = mpart

    @pl.when(hi != 0)
    def _accum():
        acc_ref[bi] = acc_ref[bi] + part
        mask_acc_ref[pl.ds(bi, 1), :] = mask_acc_ref[pl.ds(bi, 1), :] + mpart

    @pl.when(jnp.logical_and(bi == n_b - 1, hi == n_h - 1))
    def _finalize():
        total = float(n_px)
        pooled = jnp.sum(acc_ref[:], axis=(2, 3)) / total
        active = jnp.sum(mask_acc_ref[:], axis=1, keepdims=True) + 0.0001
        pooled = pooled * total / active
        z = jax.nn.sigmoid(
            jnp.dot(pooled, wt_ref[:], preferred_element_type=jnp.float32)
            + b_ref[:])
        lasso_ref[:] = jnp.full((1, 1), jnp.mean(jnp.sum(z, axis=-1)),
                                jnp.float32)

        zi = z[:, :, None]
        zj = z[:, None, :]
        ii = jax.lax.broadcasted_iota(jnp.int32, (n_b, n_ch, n_ch), 1)
        jj = jax.lax.broadcasted_iota(jnp.int32, (n_b, n_ch, n_ch), 2)
        below = jnp.logical_or(zj < zi,
                               jnp.logical_and(zj == zi, jj < ii))
        cnt = jnp.sum(below.astype(jnp.int32), axis=2)
        keep = jnp.logical_and(cnt >= k_drop, z > 0)

        n_out = n_ch * _GROUP
        row = jax.lax.broadcasted_iota(jnp.int32, (n_ch, n_out), 0)
        col = jax.lax.broadcasted_iota(jnp.int32, (n_ch, n_out), 1)
        expand = (row == col // _GROUP).astype(jnp.float32)
        out_ref[:] = jnp.dot(keep.astype(jnp.float32), expand,
                             preferred_element_type=jnp.float32
                             ).astype(jnp.int32)


def kernel(x, saliency_mask, mask_hard, W, b):
    B, C, H, Wd = saliency_mask.shape
    F = W.shape[0]
    k_drop = math.ceil((1.0 - _BUDGET) * F)

    h_blk = 32
    n_h = H // h_blk

    wt = W.T
    b2 = b.reshape(1, F)

    expanded, lasso = pl.pallas_call(
        lambda *refs: _body(*refs, n_h=n_h, n_b=B, n_ch=F, n_px=H * Wd,
                            k_drop=k_drop),
        grid=(B, n_h),
        in_specs=[
            pl.BlockSpec((1, C, h_blk, Wd), lambda bi, hi: (bi, 0, hi, 0)),
            pl.BlockSpec((1, 1, h_blk, Wd), lambda bi, hi: (bi, 0, hi, 0)),
            pl.BlockSpec((C, F), lambda bi, hi: (0, 0)),
            pl.BlockSpec((1, F), lambda bi, hi: (0, 0)),
        ],
        out_specs=[
            pl.BlockSpec((B, F * _GROUP), lambda bi, hi: (0, 0)),
            pl.BlockSpec((1, 1), lambda bi, hi: (0, 0)),
        ],
        out_shape=[
            jax.ShapeDtypeStruct((B, F * _GROUP), jnp.int32),
            jax.ShapeDtypeStruct((1, 1), jnp.float32),
        ],
        scratch_shapes=[
            pltpu.VMEM((B, C, 1, Wd), jnp.float32),
            pltpu.VMEM((B, Wd), jnp.float32),
        ],
    )(saliency_mask, mask_hard, wt, b2)

    return expanded, lasso.reshape(())


# speedup vs baseline: 1.0102x; 1.0102x over previous
import math

import jax
import jax.numpy as jnp
from jax.experimental import pallas as pl
from jax.experimental.pallas import tpu as pltpu

_GROUP = 4
_BUDGET = 0.5


def _body(sal_ref, msk_ref, wt_ref, b_ref, out_ref, lasso_ref,
          acc_ref, mask_acc_ref, *, n_c, c_blk, n_b, n_ch, n_px, k_drop):
    bi = pl.program_id(0)
    ci = pl.program_id(1)

    sal = sal_ref[0]
    m = msk_ref[0]
    part = jnp.sum(sal * m, axis=1, keepdims=True)
    acc_ref[bi, pl.ds(ci * c_blk, c_blk)] = part

    @pl.when(ci == 0)
    def _mask_sum():
        mask_acc_ref[pl.ds(bi, 1), :] = jnp.sum(m[0], axis=0, keepdims=True)

    @pl.when(jnp.logical_and(bi == n_b - 1, ci == n_c - 1))
    def _finalize():
        total = float(n_px)
        pooled = jnp.sum(acc_ref[:], axis=(2, 3)) / total
        active = jnp.sum(mask_acc_ref[:], axis=1, keepdims=True) + 0.0001
        pooled = pooled * total / active
        z = jax.nn.sigmoid(
            jnp.dot(pooled, wt_ref[:], preferred_element_type=jnp.float32)
            + b_ref[:])
        lasso_ref[:] = jnp.full((1, 1), jnp.mean(jnp.sum(z, axis=-1)),
                                jnp.float32)

        zi = z[:, :, None]
        zj = z[:, None, :]
        ii = jax.lax.broadcasted_iota(jnp.int32, (n_b, n_ch, n_ch), 1)
        jj = jax.lax.broadcasted_iota(jnp.int32, (n_b, n_ch, n_ch), 2)
        below = jnp.logical_or(zj < zi,
                               jnp.logical_and(zj == zi, jj < ii))
        cnt = jnp.sum(below.astype(jnp.int32), axis=2)
        keep = jnp.logical_and(cnt >= k_drop, z > 0)

        n_out = n_ch * _GROUP
        row = jax.lax.broadcasted_iota(jnp.int32, (n_ch, n_out), 0)
        col = jax.lax.broadcasted_iota(jnp.int32, (n_ch, n_out), 1)
        expand = (row == col // _GROUP).astype(jnp.float32)
        out_ref[:] = jnp.dot(keep.astype(jnp.float32), expand,
                             preferred_element_type=jnp.float32
                             ).astype(jnp.int32)


def kernel(x, saliency_mask, mask_hard, W, b):
    B, C, H, Wd = saliency_mask.shape
    F = W.shape[0]
    k_drop = math.ceil((1.0 - _BUDGET) * F)

    c_blk = 48
    n_c = C // c_blk

    wt = W.T
    b2 = b.reshape(1, F)

    expanded, lasso = pl.pallas_call(
        lambda *refs: _body(*refs, n_c=n_c, c_blk=c_blk, n_b=B, n_ch=F,
                            n_px=H * Wd, k_drop=k_drop),
        grid=(B, n_c),
        in_specs=[
            pl.BlockSpec((1, c_blk, H, Wd), lambda bi, ci: (bi, ci, 0, 0)),
            pl.BlockSpec((1, 1, H, Wd), lambda bi, ci: (bi, 0, 0, 0)),
            pl.BlockSpec((C, F), lambda bi, ci: (0, 0)),
            pl.BlockSpec((1, F), lambda bi, ci: (0, 0)),
        ],
        out_specs=[
            pl.BlockSpec((B, F * _GROUP), lambda bi, ci: (0, 0)),
            pl.BlockSpec((1, 1), lambda bi, ci: (0, 0)),
        ],
        out_shape=[
            jax.ShapeDtypeStruct((B, F * _GROUP), jnp.int32),
            jax.ShapeDtypeStruct((1, 1), jnp.float32),
        ],
        scratch_shapes=[
            pltpu.VMEM((B, C, 1, Wd), jnp.float32),
            pltpu.VMEM((B, Wd), jnp.float32),
        ],
    )(saliency_mask, mask_hard, wt, b2)

    return expanded, lasso.reshape(())
